# baseline (speedup 1.0000x reference)
import functools

import jax
import jax.numpy as jnp
from jax import lax
from jax.experimental import pallas as pl
from jax.experimental.pallas import tpu as pltpu
from jax.experimental.pallas import tpu_sc as plsc

EMBED_DIM = 128
VOCAB = 11
BATCH = 16384
_VPAD = 16
_NPAIR = 128
_PAIR_DIM = 2 * EMBED_DIM
_LROW = 128

_NC = 2
_NS = 16
_NW = _NC * _NS
_NPAIRS = BATCH // 2
_P_PER_W = _NPAIRS // _NW
_LANES = 16
_NCHUNK = 4
_CPAIR = _P_PER_W // _NCHUNK


def _prep_body(tpad_ref, gamma_ref, beta_ref, lab_ref, pairs_ref, pidx_ref):
    x = tpad_ref[...]
    mean = jnp.mean(x, axis=1, keepdims=True)
    d = x - mean
    var = jnp.mean(d * d, axis=1, keepdims=True)
    norm = d * lax.rsqrt(var + 1e-5) * gamma_ref[...] + beta_ref[...]

    k = lax.broadcasted_iota(jnp.int32, (_NPAIR, 1), 0)
    cols = lax.broadcasted_iota(jnp.int32, (_NPAIR, _VPAD), 1)
    oh_i = jnp.where(cols == k // VOCAB, 1.0, 0.0)
    oh_j = jnp.where(cols == k % VOCAB, 1.0, 0.0)
    pairs_ref[:, :EMBED_DIM] = jnp.dot(
        oh_i, norm, preferred_element_type=jnp.float32,
        precision=lax.Precision.HIGHEST)
    pairs_ref[:, EMBED_DIM:] = jnp.dot(
        oh_j, norm, preferred_element_type=jnp.float32,
        precision=lax.Precision.HIGHEST)

    r = lax.broadcasted_iota(jnp.int32, (_LROW, _LROW // 2), 0)
    t = lax.broadcasted_iota(jnp.int32, (_LROW, _LROW // 2), 1)
    w = jnp.where(r == 2 * t, float(VOCAB), 0.0) + jnp.where(
        r == 2 * t + 1, 1.0, 0.0)
    labf = lab_ref[...].astype(jnp.float32)
    pidx_ref[...] = jnp.dot(
        labf, w, preferred_element_type=jnp.float32,
        precision=lax.Precision.HIGHEST).astype(jnp.int32)


_tc_prep = pl.pallas_call(
    _prep_body,
    out_shape=(
        jax.ShapeDtypeStruct((_NPAIR, _PAIR_DIM), jnp.float32),
        jax.ShapeDtypeStruct((_LROW, _LROW // 2), jnp.int32),
    ),
)


_sc_mesh = plsc.VectorSubcoreMesh(core_axis_name="c", subcore_axis_name="s")


@functools.partial(
    pl.kernel,
    mesh=_sc_mesh,
    out_type=jax.ShapeDtypeStruct((_NPAIRS, _PAIR_DIM), jnp.float32),
    scratch_types=[
        pltpu.VMEM((_NPAIR, _PAIR_DIM), jnp.float32),
        pltpu.VMEM((_P_PER_W,), jnp.int32),
        pltpu.VMEM((_P_PER_W, _PAIR_DIM), jnp.float32),
        pltpu.SemaphoreType.DMA,
    ],
)
def _sc_build(pairs_hbm, pidx_hbm, out_hbm, table_v, pidx_v, rows_v, sem):
    wid = lax.axis_index("s") * _NC + lax.axis_index("c")
    base = wid * _P_PER_W
    pltpu.sync_copy(pairs_hbm, table_v)
    pltpu.sync_copy(pidx_hbm.at[pl.ds(base, _P_PER_W)], pidx_v)

    copies = []
    for ch in range(_NCHUNK):
        def body(c, _, ch=ch):
            g = ch * _CPAIR + c * _LANES
            pv = pidx_v[pl.ds(g, _LANES)]
            for l in range(_LANES):
                rows_v[g + l, :] = table_v[pv[l], :]
            return _

        lax.fori_loop(0, _CPAIR // _LANES, body, None)
        copies.append(pltpu.async_copy(
            rows_v.at[pl.ds(ch * _CPAIR, _CPAIR)],
            out_hbm.at[pl.ds(base + ch * _CPAIR, _CPAIR)],
            sem,
        ))
    for cp in copies:
        cp.wait()


def kernel(labels, table, gamma, beta):
    tpad = jnp.zeros((_VPAD, EMBED_DIM), jnp.float32).at[:VOCAB].set(table)
    lab2d = labels.astype(jnp.int32).reshape(_LROW, _LROW)
    pairs, pidx = _tc_prep(
        tpad, gamma.reshape(1, EMBED_DIM), beta.reshape(1, EMBED_DIM), lab2d
    )
    out = _sc_build(pairs, pidx.reshape(_NPAIRS))
    return out.reshape(BATCH, 1, EMBED_DIM)
